# baseline (speedup 1.0000x reference)
import functools

import jax
import jax.numpy as jnp
import numpy as np
from jax import lax
from jax.experimental import pallas as pl
from jax.experimental.pallas import tpu as pltpu
from jax.experimental.pallas import tpu_sc as plsc

_NC = 2
_NS = 16
_L = 16
_NW = _NC * _NS

_C = 64
_G = 8
_NCH = 160
_NPAD = 10240
_MASK = np.int32(-65536)


def _perm128():
    p = []
    for g in range(4):
        for k in range(16):
            p.extend([32 * g + k, 32 * g + 16 + k])
    return p


def _edge_mm_body(ea_ref, w_ref, b_ref, out_ref):
    acc = (jnp.dot(ea_ref[...], w_ref[...],
                   preferred_element_type=jnp.float32) + b_ref[...])
    out_ref[...] = acc.astype(jnp.bfloat16)


def _edge_project(ea2, W2, b2, EPP):
    BE = 2048
    grid = EPP // BE
    return pl.pallas_call(
        _edge_mm_body,
        grid=(grid,),
        in_specs=[
            pl.BlockSpec((BE, 32), lambda i: (i, 0)),
            pl.BlockSpec((32, 256), lambda i: (0, 0)),
            pl.BlockSpec((1, 256), lambda i: (0, 0)),
        ],
        out_specs=pl.BlockSpec((BE, 256), lambda i: (i, 0)),
        out_shape=jax.ShapeDtypeStruct((EPP, 256), jnp.bfloat16),
    )(ea2, W2, b2)


def _decode(v):
    lo = plsc.bitcast(lax.shift_left(v, 16), jnp.float32)
    hi = plsc.bitcast(lax.bitwise_and(v, _MASK), jnp.float32)
    return lo, hi


def _make_sc_agg(N, D, EP):
    RT = _NPAD // _NS
    HW = D // 2
    CP = _C // 2
    mesh = plsc.VectorSubcoreMesh(core_axis_name="c", subcore_axis_name="s",
                                  num_cores=_NC, num_subcores=_NS)

    @functools.partial(
        pl.kernel,
        mesh=mesh,
        compiler_params=pltpu.CompilerParams(needs_layout_passes=False),
        out_type=jax.ShapeDtypeStruct((_NC * _NPAD, D), jnp.float32),
        scratch_types=[
            pltpu.VMEM((_G, _C), jnp.int32),
            pltpu.VMEM((_G, _C), jnp.int32),
            pltpu.VMEM((3, _C, D), jnp.float32),
            pltpu.VMEM((3, CP, D), jnp.int32),
            pltpu.VMEM_SHARED((_NPAD, D), jnp.float32),
            pltpu.SemaphoreType.DMA,
            pltpu.SemaphoreType.DMA,
            pltpu.SemaphoreType.DMA,
            pltpu.SemaphoreType.DMA,
            pltpu.SemaphoreType.DMA,
            pltpu.SemaphoreType.DMA,
            pltpu.SemaphoreType.DMA,
            pltpu.SemaphoreType.DMA,
            pltpu.SemaphoreType.DMA,
        ],
    )
    def sc_agg(x_hbm, srcb_hbm, dstb_hbm, e_hbm, out_hbm,
               idxs, idxd, xv, ei, agg,
               se0, se1, se2, sg0, sg1, sg2, ss0, ss1, ss2):
        sems_e = (se0, se1, se2)
        sems_g = (sg0, sg1, sg2)
        sems_s = (ss0, ss1, ss2)
        c = lax.axis_index("c")
        s = lax.axis_index("s")
        wid = c * _NS + s
        ibase = wid * _NCH

        @plsc.parallel_loop(0, _C, unroll=4)
        def zrow(r):
            for cc in range(D // _L):
                xv[0, r, pl.ds(cc * _L, _L)] = jnp.zeros((_L,), jnp.float32)

        for k in range(RT // _C):
            pltpu.sync_copy(
                xv.at[0],
                agg.at[pl.ds(pl.multiple_of(s * RT + k * _C, 8), _C)])
        plsc.subcore_barrier()

        def issue_loads(g, k):
            b = k % 3
            erow = pl.multiple_of((ibase + g * _G + k) * CP, 8)
            de = pltpu.async_copy(e_hbm.at[pl.ds(erow, CP)], ei.at[b],
                                  sems_e[b])
            dg = pltpu.async_copy(x_hbm.at[idxs.at[k]], xv.at[b], sems_g[b])
            return de, dg

        def group(g, carry):
            row0 = pl.multiple_of(ibase + g * _G, 8)
            pltpu.sync_copy(srcb_hbm.at[pl.ds(row0, _G)], idxs)
            pltpu.sync_copy(dstb_hbm.at[pl.ds(row0, _G)], idxd)

            loads = {0: issue_loads(g, 0), 1: issue_loads(g, 1)}
            scat = {}
            for k in range(_G):
                b = k % 3
                de, dg = loads.pop(k)
                de.wait()
                dg.wait()

                @plsc.parallel_loop(0, CP, unroll=1)
                def rowfn(rp):
                    for a in range(2):
                        r = 2 * rp + a
                        for g4 in range(HW // _L):
                            ew = ei[b, rp, pl.ds(a * HW + g4 * _L, _L)]
                            elo, ehi = _decode(ew)
                            slo = pl.ds(2 * g4 * _L, _L)
                            shi = pl.ds((2 * g4 + 1) * _L, _L)
                            xv[b, r, slo] = jnp.maximum(
                                xv[b, r, slo] + elo, 0.0)
                            xv[b, r, shi] = jnp.maximum(
                                xv[b, r, shi] + ehi, 0.0)

                scat[k] = pltpu.async_copy(
                    xv.at[b], agg.at[idxd.at[k]], sems_s[b], add=True)
                if k + 2 < _G:
                    if k >= 1:
                        scat.pop(k - 1).wait()
                    loads[k + 2] = issue_loads(g, k + 2)
            for k in sorted(scat):
                scat.pop(k).wait()
            return carry

        lax.fori_loop(0, _NCH // _G, group, 0)
        plsc.subcore_barrier()

        pltpu.sync_copy(
            agg.at[pl.ds(pl.multiple_of(s * RT, 8), RT)],
            out_hbm.at[pl.ds(pl.multiple_of(c * _NPAD + s * RT, 8), RT)])

    return sc_agg


def _mlp_body(x_ref, p0_ref, p1_ref, w1_ref, b1_ref, w2_ref,
              b2_ref, g_ref, bb_ref, o_ref):
    h = x_ref[...] + p0_ref[0] + p1_ref[0]
    t = jnp.maximum(
        jnp.dot(h, w1_ref[...], preferred_element_type=jnp.float32)
        + b1_ref[...], 0.0)
    h2 = (jnp.dot(t, w2_ref[...], preferred_element_type=jnp.float32)
          + b2_ref[...])
    mu = jnp.mean(h2, axis=-1, keepdims=True)
    var = jnp.mean((h2 - mu) ** 2, axis=-1, keepdims=True)
    hn = (h2 - mu) * lax.rsqrt(var + 1e-5) * g_ref[...] + bb_ref[...]
    o_ref[...] = jnp.maximum(hn, 0.0)


def _node_update(x, parts3, W1, b1, W2, b2, gamma, beta):
    N, D = x.shape
    BN = 2000
    grid = N // BN
    full = lambda i: (0, 0)
    return pl.pallas_call(
        _mlp_body,
        grid=(grid,),
        in_specs=[
            pl.BlockSpec((BN, D), lambda i: (i, 0)),
            pl.BlockSpec((1, BN, D), lambda i: (0, i, 0)),
            pl.BlockSpec((1, BN, D), lambda i: (1, i, 0)),
            pl.BlockSpec((D, D), full),
            pl.BlockSpec((1, D), full),
            pl.BlockSpec((D, D), full),
            pl.BlockSpec((1, D), full),
            pl.BlockSpec((1, D), full),
            pl.BlockSpec((1, D), full),
        ],
        out_specs=pl.BlockSpec((BN, D), lambda i: (i, 0)),
        out_shape=jax.ShapeDtypeStruct((N, D), jnp.float32),
    )(x, parts3, parts3, W1, b1.reshape(1, D), W2, b2.reshape(1, D),
      gamma.reshape(1, D), beta.reshape(1, D))


def kernel(x, edge_index, edge_attr, We, be, W1, b1, W2, b2, gamma, beta):
    N, D = x.shape
    E, ED = edge_attr.shape
    EP = _NW * _NCH * _C
    EPP = EP // 2
    pad = EP - E

    col_at_pos = np.array(_perm128(), np.int32)

    Wep = We[:, col_at_pos]
    bep = be[col_at_pos]

    z = jnp.zeros((ED, D), jnp.float32)
    W2p = jnp.block([[Wep, z], [z, Wep]])
    b2p = jnp.concatenate([bep, bep]).reshape(1, 256)

    ea2 = jnp.concatenate(
        [edge_attr, jnp.zeros((pad, ED), jnp.float32)]).reshape(EPP, 2 * ED)
    e_bf = _edge_project(ea2, W2p, b2p, EPP)
    e_i32 = lax.bitcast_convert_type(
        e_bf.reshape(EPP, 128, 2), jnp.int32)

    src2d = jnp.concatenate(
        [edge_index[0], jnp.zeros((pad,), jnp.int32)]).reshape(EP // _C, _C)
    pad_dst = N + jnp.arange(pad, dtype=jnp.int32) % (_NPAD - N)
    dst2d = jnp.concatenate(
        [edge_index[1], pad_dst]).reshape(EP // _C, _C)

    parts = _make_sc_agg(N, D, EP)(x, src2d, dst2d, e_i32)
    parts3 = parts.reshape(_NC, _NPAD, D)
    return _node_update(x, parts3, W1, b1, W2, b2, gamma, beta)


# speedup vs baseline: 1.7749x; 1.7749x over previous
import functools

import jax
import jax.numpy as jnp
from jax import lax
from jax.experimental import pallas as pl
from jax.experimental.pallas import tpu as pltpu
from jax.experimental.pallas import tpu_sc as plsc

_NC = 2
_NS = 16
_L = 16
_NW = _NC * _NS

_C = 128
_KC = 80
_KCB = 8


def _edge_mm_body(ea_ref, we_ref, be_ref, out_ref):
    out_ref[...] = (
        jnp.dot(ea_ref[...], we_ref[...], preferred_element_type=jnp.float32)
        + be_ref[...]
    )


def _edge_project(edge_attr, We, be, e_rows):
    E, ED = edge_attr.shape
    D = We.shape[1]
    BE = 4000
    grid = E // BE
    return pl.pallas_call(
        _edge_mm_body,
        grid=(grid,),
        in_specs=[
            pl.BlockSpec((BE, ED), lambda i: (i, 0)),
            pl.BlockSpec((ED, D), lambda i: (0, 0)),
            pl.BlockSpec((1, D), lambda i: (0, 0)),
        ],
        out_specs=pl.BlockSpec((BE, D), lambda i: (i, 0)),
        out_shape=jax.ShapeDtypeStruct((e_rows, D), jnp.float32),
    )(edge_attr, We, be.reshape(1, D))


def _make_sc_agg(N, NP, D):
    RPT = NP // _NS
    KOUT = RPT // _C
    mesh = plsc.VectorSubcoreMesh(core_axis_name="c", subcore_axis_name="s",
                                  num_cores=_NC, num_subcores=_NS)

    @functools.partial(
        pl.kernel,
        mesh=mesh,
        out_type=jax.ShapeDtypeStruct((_NC * NP, D), jnp.float32),
        scratch_types=[
            pltpu.VMEM((_KCB, _C), jnp.int32),
            pltpu.VMEM((_KCB, _C), jnp.int32),
            pltpu.VMEM((_C, D), jnp.float32),
            pltpu.VMEM((_C, D), jnp.float32),
            pltpu.VMEM_SHARED((NP, D), jnp.float32),
            pltpu.SemaphoreType.DMA,
        ],
    )
    def sc_agg(x_hbm, src_hbm, dst_hbm, e_hbm, out_hbm,
               src_v, dst_v, xv, ev, agg, sem):
        c = lax.axis_index("c")
        s = lax.axis_index("s")
        wid = c * _NS + s

        def zrow(r, carry):
            for cc in range(D // _L):
                xv[r, pl.ds(cc * _L, _L)] = jnp.zeros((_L,), jnp.float32)
            return carry

        lax.fori_loop(0, _C, zrow, 0)
        for k in range(KOUT):
            pltpu.sync_copy(
                xv, agg.at[pl.ds(pl.multiple_of(s * RPT + k * _C, 8), _C)])
        plsc.subcore_barrier()

        def blk(jj, carry):
            ibase = pl.multiple_of(wid * _KC + jj * _KCB, 8)
            pltpu.sync_copy(src_hbm.at[pl.ds(ibase, _KCB)], src_v)
            pltpu.sync_copy(dst_hbm.at[pl.ds(ibase, _KCB)], dst_v)

            def chunk(jb, carry1):
                erow = pl.multiple_of(
                    (wid * _KC + jj * _KCB + jb) * _C, 8)
                pltpu.sync_copy(e_hbm.at[pl.ds(erow, _C)], ev)
                pltpu.async_copy(x_hbm.at[src_v.at[jb]], xv, sem).wait()

                def row(r, carry2):
                    for cc in range(D // _L):
                        sl = pl.ds(cc * _L, _L)
                        xv[r, sl] = jnp.maximum(xv[r, sl] + ev[r, sl], 0.0)
                    return carry2

                lax.fori_loop(0, _C, row, 0)
                pltpu.sync_copy(xv, agg.at[dst_v.at[jb]], add=True)
                return carry1

            lax.fori_loop(0, _KCB, chunk, 0)
            return carry

        lax.fori_loop(0, _KC // _KCB, blk, 0)
        plsc.subcore_barrier()

        for k in range(KOUT):
            off = pl.multiple_of(s * RPT + k * _C, 8)
            pltpu.sync_copy(agg.at[pl.ds(off, _C)],
                            out_hbm.at[pl.ds(pl.multiple_of(c * NP, 8) + off,
                                             _C)])

    return sc_agg


def _mlp_body(x_ref, p0_ref, p1_ref, w1_ref, b1_ref, w2_ref, b2_ref,
              g_ref, bb_ref, o_ref):
    h = x_ref[...] + p0_ref[0] + p1_ref[0]
    t = jnp.maximum(
        jnp.dot(h, w1_ref[...], preferred_element_type=jnp.float32)
        + b1_ref[...], 0.0)
    h2 = (jnp.dot(t, w2_ref[...], preferred_element_type=jnp.float32)
          + b2_ref[...])
    mu = jnp.mean(h2, axis=-1, keepdims=True)
    var = jnp.mean((h2 - mu) ** 2, axis=-1, keepdims=True)
    hn = (h2 - mu) * lax.rsqrt(var + 1e-5) * g_ref[...] + bb_ref[...]
    o_ref[...] = jnp.maximum(hn, 0.0)


def _node_update(x, parts3, W1, b1, W2, b2, gamma, beta):
    N, D = x.shape
    BN = 2000
    grid = N // BN
    full = lambda i: (0, 0)
    return pl.pallas_call(
        _mlp_body,
        grid=(grid,),
        in_specs=[
            pl.BlockSpec((BN, D), lambda i: (i, 0)),
            pl.BlockSpec((1, BN, D), lambda i: (0, i, 0)),
            pl.BlockSpec((1, BN, D), lambda i: (1, i, 0)),
            pl.BlockSpec((D, D), full),
            pl.BlockSpec((1, D), full),
            pl.BlockSpec((D, D), full),
            pl.BlockSpec((1, D), full),
            pl.BlockSpec((1, D), full),
            pl.BlockSpec((1, D), full),
        ],
        out_specs=pl.BlockSpec((BN, D), lambda i: (i, 0)),
        out_shape=jax.ShapeDtypeStruct((N, D), jnp.float32),
    )(x, parts3, parts3, W1, b1.reshape(1, D), W2, b2.reshape(1, D),
      gamma.reshape(1, D), beta.reshape(1, D))


def kernel(x, edge_index, edge_attr, We, be, W1, b1, W2, b2, gamma, beta):
    N, D = x.shape
    E = edge_attr.shape[0]
    EP = _NW * _KC * _C
    NP = ((N // _NS) // _C + 1) * _C * _NS
    pad = EP - E
    e = _edge_project(edge_attr, We, be, EP)
    src2d = jnp.concatenate(
        [edge_index[0], jnp.zeros((pad,), jnp.int32)]).reshape(EP // _C, _C)
    pad_dst = N + jnp.arange(pad, dtype=jnp.int32) % (NP - N)
    dst2d = jnp.concatenate(
        [edge_index[1], pad_dst]).reshape(EP // _C, _C)
    parts = _make_sc_agg(N, NP, D)(x, src2d, dst2d, e)
    parts3 = parts.reshape(_NC, NP, D)
    return _node_update(x, parts3, W1, b1, W2, b2, gamma, beta)


# speedup vs baseline: 1.9009x; 1.0710x over previous
import functools

import jax
import jax.numpy as jnp
from jax import lax
from jax.experimental import pallas as pl
from jax.experimental.pallas import tpu as pltpu
from jax.experimental.pallas import tpu_sc as plsc

_NC = 2
_NS = 16
_L = 16
_NW = _NC * _NS

_C = 128
_KC = 80
_KCB = 8


def _edge_mm_body(ea_ref, we_ref, be_ref, out_ref):
    out_ref[...] = (
        jnp.dot(ea_ref[...], we_ref[...], preferred_element_type=jnp.float32)
        + be_ref[...]
    )


def _edge_project(edge_attr, We, be, e_rows):
    E, ED = edge_attr.shape
    D = We.shape[1]
    BE = 4096
    grid = e_rows // BE
    return pl.pallas_call(
        _edge_mm_body,
        grid=(grid,),
        in_specs=[
            pl.BlockSpec((BE, ED), lambda i: (i, 0)),
            pl.BlockSpec((ED, D), lambda i: (0, 0)),
            pl.BlockSpec((1, D), lambda i: (0, 0)),
        ],
        out_specs=pl.BlockSpec((BE, D), lambda i: (i, 0)),
        out_shape=jax.ShapeDtypeStruct((e_rows, D), jnp.float32),
    )(edge_attr, We, be.reshape(1, D))


def _make_sc_agg(N, NP, D, KC):
    RPT = NP // _NS
    KOUT = RPT // _C
    mesh = plsc.VectorSubcoreMesh(core_axis_name="c", subcore_axis_name="s",
                                  num_cores=_NC, num_subcores=_NS)

    @functools.partial(
        pl.kernel,
        mesh=mesh,
        out_type=jax.ShapeDtypeStruct((_NC * NP, D), jnp.float32),
        scratch_types=[
            pltpu.VMEM((_KCB, _C), jnp.int32),
            pltpu.VMEM((_KCB, _C), jnp.int32),
            pltpu.VMEM((_C, D), jnp.float32),
            pltpu.VMEM((_C, D), jnp.float32),
            pltpu.VMEM_SHARED((NP, D), jnp.float32),
            pltpu.SemaphoreType.DMA,
        ],
    )
    def sc_agg(x_hbm, src_hbm, dst_hbm, e_hbm, out_hbm,
               src_v, dst_v, xv, ev, agg, sem):
        c = lax.axis_index("c")
        s = lax.axis_index("s")
        wid = c * _NS + s

        def zrow(r, carry):
            for cc in range(D // _L):
                xv[r, pl.ds(cc * _L, _L)] = jnp.zeros((_L,), jnp.float32)
            return carry

        lax.fori_loop(0, _C, zrow, 0)
        for k in range(KOUT):
            pltpu.sync_copy(
                xv, agg.at[pl.ds(pl.multiple_of(s * RPT + k * _C, 8), _C)])
        plsc.subcore_barrier()

        def blk(jj, carry):
            ibase = pl.multiple_of(wid * KC + jj * _KCB, 8)
            pltpu.sync_copy(src_hbm.at[pl.ds(ibase, _KCB)], src_v)
            pltpu.sync_copy(dst_hbm.at[pl.ds(ibase, _KCB)], dst_v)

            def chunk(jb, carry1):
                erow = pl.multiple_of(
                    (wid * KC + jj * _KCB + jb) * _C, 8)
                pltpu.sync_copy(e_hbm.at[pl.ds(erow, _C)], ev)
                pltpu.async_copy(x_hbm.at[src_v.at[jb]], xv, sem).wait()

                def row(r, carry2):
                    for cc in range(D // _L):
                        sl = pl.ds(cc * _L, _L)
                        xv[r, sl] = jnp.maximum(xv[r, sl] + ev[r, sl], 0.0)
                    return carry2

                lax.fori_loop(0, _C, row, 0)
                pltpu.sync_copy(xv, agg.at[dst_v.at[jb]], add=True)
                return carry1

            lax.fori_loop(0, _KCB, chunk, 0)
            return carry

        lax.fori_loop(0, KC // _KCB, blk, 0)
        plsc.subcore_barrier()

        for k in range(KOUT):
            off = pl.multiple_of(s * RPT + k * _C, 8)
            pltpu.sync_copy(agg.at[pl.ds(off, _C)],
                            out_hbm.at[pl.ds(pl.multiple_of(c * NP, 8) + off,
                                             _C)])

    return sc_agg


def _mlp_body(x_ref, pa0_ref, pa1_ref, pb0_ref, pb1_ref, w1_ref, b1_ref,
              w2_ref, b2_ref, g_ref, bb_ref, o_ref):
    h = (x_ref[...] + pa0_ref[0] + pa1_ref[0] + pb0_ref[0] + pb1_ref[0])
    t = jnp.maximum(
        jnp.dot(h, w1_ref[...], preferred_element_type=jnp.float32)
        + b1_ref[...], 0.0)
    h2 = (jnp.dot(t, w2_ref[...], preferred_element_type=jnp.float32)
          + b2_ref[...])
    mu = jnp.mean(h2, axis=-1, keepdims=True)
    var = jnp.mean((h2 - mu) ** 2, axis=-1, keepdims=True)
    hn = (h2 - mu) * lax.rsqrt(var + 1e-5) * g_ref[...] + bb_ref[...]
    o_ref[...] = jnp.maximum(hn, 0.0)


def _node_update(x, pa3, pb3, W1, b1, W2, b2, gamma, beta):
    N, D = x.shape
    BN = 2000
    grid = N // BN
    full = lambda i: (0, 0)
    return pl.pallas_call(
        _mlp_body,
        grid=(grid,),
        in_specs=[
            pl.BlockSpec((BN, D), lambda i: (i, 0)),
            pl.BlockSpec((1, BN, D), lambda i: (0, i, 0)),
            pl.BlockSpec((1, BN, D), lambda i: (1, i, 0)),
            pl.BlockSpec((1, BN, D), lambda i: (0, i, 0)),
            pl.BlockSpec((1, BN, D), lambda i: (1, i, 0)),
            pl.BlockSpec((D, D), full),
            pl.BlockSpec((1, D), full),
            pl.BlockSpec((D, D), full),
            pl.BlockSpec((1, D), full),
            pl.BlockSpec((1, D), full),
            pl.BlockSpec((1, D), full),
        ],
        out_specs=pl.BlockSpec((BN, D), lambda i: (i, 0)),
        out_shape=jax.ShapeDtypeStruct((N, D), jnp.float32),
    )(x, pa3, pa3, pb3, pb3, W1, b1.reshape(1, D), W2, b2.reshape(1, D),
      gamma.reshape(1, D), beta.reshape(1, D))


def kernel(x, edge_index, edge_attr, We, be, W1, b1, W2, b2, gamma, beta):
    N, D = x.shape
    E = edge_attr.shape[0]
    EP = _NW * _KC * _C
    EPH = EP // 2
    KCH = _KC // 2
    NP = ((N // _NS) // _C + 1) * _C * _NS
    pad = EP - E

    eaA = edge_attr[:EPH]
    eaB = jnp.concatenate([edge_attr[EPH:],
                           jnp.zeros((pad, edge_attr.shape[1]), jnp.float32)])
    srcA = edge_index[0][:EPH].reshape(EPH // _C, _C)
    dstA = edge_index[1][:EPH].reshape(EPH // _C, _C)
    srcB = jnp.concatenate(
        [edge_index[0][EPH:], jnp.zeros((pad,), jnp.int32)]
    ).reshape(EPH // _C, _C)
    pad_dst = N + jnp.arange(pad, dtype=jnp.int32) % (NP - N)
    dstB = jnp.concatenate(
        [edge_index[1][EPH:], pad_dst]).reshape(EPH // _C, _C)

    sc = _make_sc_agg(N, NP, D, KCH)
    eA = _edge_project(eaA, We, be, EPH)
    pA = sc(x, srcA, dstA, eA)
    eB = _edge_project(eaB, We, be, EPH)
    pB = sc(x, srcB, dstB, eB)
    return _node_update(x, pA.reshape(_NC, NP, D), pB.reshape(_NC, NP, D),
                        W1, b1, W2, b2, gamma, beta)
